# baseline (speedup 1.0000x reference)
import functools

import jax
import jax.numpy as jnp
from jax import lax
from jax.experimental import pallas as pl
from jax.experimental.pallas import tpu as pltpu
from jax.experimental.pallas import tpu_sc as plsc

L = 16
NTILES = 16
NB = 16
NR = 4
ZR = 16


def _mlp_pallas(s, W1t, b1, W2t, b2):
    N, H = s.shape
    TH = W2t.shape[1]
    R = 400
    assert N % R == 0

    def body(s_ref, w1_ref, b1_ref, w2_ref, b2_ref, h0_ref, h12_ref):
        t = jnp.dot(s_ref[...], w1_ref[...], preferred_element_type=jnp.float32)
        t = jnp.maximum(t + b1_ref[...], 0.0)
        hh = jnp.dot(t, w2_ref[...], preferred_element_type=jnp.float32)
        hh = hh + b2_ref[...]
        h0_ref[...] = hh[:, :H]
        h12_ref[...] = hh[:, H:]

    return pl.pallas_call(
        body,
        grid=(N // R,),
        in_specs=[
            pl.BlockSpec((R, H), lambda i: (i, 0)),
            pl.BlockSpec((H, H), lambda i: (0, 0)),
            pl.BlockSpec((1, H), lambda i: (0, 0)),
            pl.BlockSpec((H, TH), lambda i: (0, 0)),
            pl.BlockSpec((1, TH), lambda i: (0, 0)),
        ],
        out_specs=[
            pl.BlockSpec((R, H), lambda i: (i, 0)),
            pl.BlockSpec((R, TH - H), lambda i: (i, 0)),
        ],
        out_shape=[
            jax.ShapeDtypeStruct((N, H), jnp.float32),
            jax.ShapeDtypeStruct((N, TH - H), jnp.float32),
        ],
    )(s, W1t, b1, W2t, b2)


def _filter_pallas(edge_dist, Wft, bf):
    E, G = edge_dist.shape
    TH = Wft.shape[1]
    H = TH // 3
    R = 2000
    assert E % R == 0

    def body(d_ref, wf_ref, bf_ref, w0_ref, w12_ref):
        ww = jnp.dot(d_ref[...], wf_ref[...], preferred_element_type=jnp.float32)
        ww = ww + bf_ref[...]
        w0_ref[...] = ww[:, :H]
        w12_ref[...] = ww[:, H:]

    return pl.pallas_call(
        body,
        grid=(E // R,),
        in_specs=[
            pl.BlockSpec((R, G), lambda i: (i, 0)),
            pl.BlockSpec((G, TH), lambda i: (0, 0)),
            pl.BlockSpec((1, TH), lambda i: (0, 0)),
        ],
        out_specs=[
            pl.BlockSpec((R, H), lambda i: (i, 0)),
            pl.BlockSpec((R, TH - H), lambda i: (i, 0)),
        ],
        out_shape=[
            jax.ShapeDtypeStruct((E, H), jnp.float32),
            jax.ShapeDtypeStruct((E, TH - H), jnp.float32),
        ],
    )(edge_dist, Wft, bf)


def _add_pallas(a, b):
    N, H = a.shape
    R = 400

    def body(a_ref, b_ref, o_ref):
        o_ref[...] = a_ref[...] + b_ref[...]

    return pl.pallas_call(
        body,
        grid=(N // R,),
        in_specs=[pl.BlockSpec((R, H), lambda i: (i, 0)),
                  pl.BlockSpec((R, H), lambda i: (i, 0))],
        out_specs=pl.BlockSpec((R, H), lambda i: (i, 0)),
        out_shape=jax.ShapeDtypeStruct((N, H), jnp.float32),
    )(a, b)


@functools.cache
def _make_sc_kernel(N, E, H):
    assert N % ZR == 0
    assert E % (NTILES * NB) == 0
    nch = N // ZR
    nch_pt = -(-nch // NTILES)
    ept = E // NTILES
    nbatch = ept // NB
    E1A = (3 * E) // 10
    assert E1A % (NTILES * NB) == 0 and (E - E1A) % (NTILES * NB) == 0
    nslc = H // L
    mesh = plsc.VectorSubcoreMesh(core_axis_name="c", subcore_axis_name="s",
                                  num_cores=2, num_subcores=NTILES)

    ring_types = [
        pltpu.VMEM((NB,), jnp.int32),
        pltpu.VMEM((NB,), jnp.int32),
        pltpu.VMEM((NB,), jnp.float32),
        pltpu.VMEM((NB, 2 * H), jnp.float32),
        pltpu.VMEM((NB, 2 * H), jnp.float32),
        pltpu.VMEM((NB, H), jnp.float32),
        pltpu.SemaphoreType.DMA,
        pltpu.SemaphoreType.DMA,
    ]

    @functools.partial(
        pl.kernel,
        out_type=[jax.ShapeDtypeStruct((N, H), jnp.float32)] * 5,
        mesh=mesh,
        scratch_types=[
            pltpu.VMEM_SHARED((N, H), jnp.float32),
            pltpu.VMEM((ZR, H), jnp.float32),
        ] + ring_types * NR,
    )
    def sc_kernel(h0, h12, w0, w12, v0, v1, v2, ev0, ev1, ev2, src, dst,
                  s_out, u0_out, u1a_out, u1b_out, u2_out,
                  acc, zbuf, *ringargs):
        sets = [ringargs[i * 8:(i + 1) * 8] for i in range(NR)]
        cid = lax.axis_index("c")
        sid = lax.axis_index("s")
        zero16 = jnp.zeros((L,), jnp.float32)

        def fill_zbuf(i, c):
            for j in range(nslc):
                zbuf[i, pl.ds(L * j, L)] = zero16
            return c

        def zero_acc():
            lax.fori_loop(0, ZR, fill_zbuf, 0)

            def z(i, c):
                cidx = sid + NTILES * i

                @pl.when(cidx < nch)
                def _():
                    pltpu.sync_copy(zbuf, acc.at[pl.ds(cidx * ZR, ZR)])
                return c
            lax.fori_loop(0, nch_pt, z, 0)

        def writeout(out_hbm):
            def wlp(i, c):
                cidx = sid + NTILES * i

                @pl.when(cidx < nch)
                def _():
                    r0 = cidx * ZR
                    pltpu.sync_copy(acc.at[pl.ds(r0, ZR)], zbuf)
                    pltpu.sync_copy(zbuf, out_hbm.at[pl.ds(r0, ZR)])
                return c
            lax.fori_loop(0, nch_pt, wlp, 0)

        def pipelined(nb_total, issue_small, wait_small, issue_big, wait_big,
                      do_batch):
            for j in range(NR):
                if j < nb_total:
                    issue_small(j, j)
            for j in range(NR - 1):
                if j < nb_total:
                    wait_small(j, j)
                    issue_big(j, j)

            def phase(b, slot):
                bn = b + NR - 1
                nslot = (slot + NR - 1) % NR

                @pl.when(bn < nb_total)
                def _():
                    wait_small(bn, nslot)
                    issue_big(bn, nslot)
                wait_big(b, slot)
                do_batch(b, slot)

                @pl.when(b + NR < nb_total)
                def _():
                    issue_small(b + NR, slot)

            def grp(i, c):
                b0 = NR * i
                phase(b0, 0)
                for p in range(1, NR):
                    @pl.when(b0 + p < nb_total)
                    def _(p=p):
                        phase(b0 + p, p)
                return c

            lax.fori_loop(0, -(-nb_total // NR), grp, 0)

        def accum_s():
            def issue_small(b, k):
                srcb, dstb, semS = sets[k][0], sets[k][1], sets[k][6]
                base = sid * ept + b * NB
                pltpu.async_copy(dst.at[pl.ds(base, NB)], dstb, semS)
                pltpu.async_copy(src.at[pl.ds(base, NB)], srcb, semS)

            def wait_small(b, k):
                srcb, dstb, semS = sets[k][0], sets[k][1], sets[k][6]
                base = sid * ept + b * NB
                pltpu.make_async_copy(dst.at[pl.ds(base, NB)], dstb, semS).wait()
                pltpu.make_async_copy(src.at[pl.ds(base, NB)], srcb, semS).wait()

            def issue_big(b, k):
                srcb, semB = sets[k][0], sets[k][7]
                vkbuf, wk = sets[k][5], sets[k + NR // 2][5]
                base = sid * ept + b * NB
                pltpu.async_copy(h0.at[srcb], vkbuf, semB)
                pltpu.async_copy(w0.at[pl.ds(base, NB)], wk, semB)

            def wait_big(b, k):
                srcb, semB = sets[k][0], sets[k][7]
                vkbuf, wk = sets[k][5], sets[k + NR // 2][5]
                base = sid * ept + b * NB
                pltpu.make_async_copy(h0.at[srcb], vkbuf, semB).wait()
                pltpu.make_async_copy(w0.at[pl.ds(base, NB)], wk, semB).wait()

            def do_batch(b, k):
                dstb = sets[k][1]
                vkbuf, wk = sets[k][5], sets[k + NR // 2][5]

                def edge(e, c2):
                    for j in range(nslc):
                        sl = pl.ds(L * j, L)
                        vkbuf[e, sl] = vkbuf[e, sl] * wk[e, sl]
                    return c2

                lax.fori_loop(0, NB, edge, 0)
                pltpu.sync_copy(vkbuf, acc.at[dstb], add=True)

            def pipelined_s():
                nrs = NR // 2
                for j in range(nrs):
                    issue_small(j, j)
                for j in range(nrs - 1):
                    wait_small(j, j)
                    issue_big(j, j)

                def phase(b, slot):
                    bn = b + nrs - 1
                    nslot = (slot + nrs - 1) % nrs

                    @pl.when(bn < nbatch)
                    def _():
                        wait_small(bn, nslot)
                        issue_big(bn, nslot)
                    wait_big(b, slot)
                    do_batch(b, slot)

                    @pl.when(b + nrs < nbatch)
                    def _():
                        issue_small(b + nrs, slot)

                def grp(i, c):
                    b0 = nrs * i
                    phase(b0, 0)
                    for p in range(1, nrs):
                        @pl.when(b0 + p < nbatch)
                        def _(p=p):
                            phase(b0 + p, p)
                    return c

                lax.fori_loop(0, -(-nbatch // nrs), grp, 0)

            pipelined_s()

        def accum_v(vk, evk, ebase, ecnt):
            ept_c = ecnt // NTILES
            nbatch_c = ept_c // NB

            def issue_small(b, k):
                srcb, dstb, evb = sets[k][0], sets[k][1], sets[k][2]
                semS = sets[k][6]
                base = ebase + sid * ept_c + b * NB
                pltpu.async_copy(dst.at[pl.ds(base, NB)], dstb, semS)
                pltpu.async_copy(src.at[pl.ds(base, NB)], srcb, semS)
                pltpu.async_copy(evk.at[pl.ds(base, NB)], evb, semS)

            def wait_small(b, k):
                srcb, dstb, evb = sets[k][0], sets[k][1], sets[k][2]
                semS = sets[k][6]
                base = ebase + sid * ept_c + b * NB
                pltpu.make_async_copy(dst.at[pl.ds(base, NB)], dstb, semS).wait()
                pltpu.make_async_copy(src.at[pl.ds(base, NB)], srcb, semS).wait()
                pltpu.make_async_copy(evk.at[pl.ds(base, NB)], evb, semS).wait()

            def issue_big(b, k):
                srcb, hbuf, wbuf, vkbuf = (sets[k][0], sets[k][3], sets[k][4],
                                           sets[k][5])
                semB = sets[k][7]
                base = ebase + sid * ept_c + b * NB
                pltpu.async_copy(h12.at[srcb], hbuf, semB)
                pltpu.async_copy(vk.at[srcb], vkbuf, semB)
                pltpu.async_copy(w12.at[pl.ds(base, NB)], wbuf, semB)

            def wait_big(b, k):
                srcb, hbuf, wbuf, vkbuf = (sets[k][0], sets[k][3], sets[k][4],
                                           sets[k][5])
                semB = sets[k][7]
                base = ebase + sid * ept_c + b * NB
                pltpu.make_async_copy(h12.at[srcb], hbuf, semB).wait()
                pltpu.make_async_copy(vk.at[srcb], vkbuf, semB).wait()
                pltpu.make_async_copy(w12.at[pl.ds(base, NB)], wbuf, semB).wait()

            def do_batch(b, k):
                dstb, evb, hbuf, wbuf, vkbuf = (sets[k][1], sets[k][2],
                                                sets[k][3], sets[k][4],
                                                sets[k][5])

                def edge_group(g, c2):
                    ev16 = evb[pl.ds(L * g, L)]
                    evs = [ev16[t] for t in range(L)]

                    def jbody(j, c3):
                        sl = pl.ds(L * j, L)
                        sl2 = pl.ds(H + L * j, L)
                        for t in range(L):
                            e = L * g + t
                            g1 = hbuf[e, sl] * wbuf[e, sl]
                            g2 = hbuf[e, sl2] * wbuf[e, sl2]
                            vkbuf[e, sl] = vkbuf[e, sl] * g1 + evs[t] * g2
                        return c3

                    lax.fori_loop(0, nslc, jbody, 0)
                    return c2

                lax.fori_loop(0, NB // L, edge_group, 0)
                pltpu.sync_copy(vkbuf, acc.at[dstb], add=True)

            pipelined(nbatch_c, issue_small, wait_small, issue_big, wait_big,
                      do_batch)

        def chunk(accum_fn, out_hbm):
            zero_acc()
            plsc.subcore_barrier()
            accum_fn()
            plsc.subcore_barrier()
            writeout(out_hbm)
            plsc.subcore_barrier()

        @pl.when(cid == 0)
        def _():
            chunk(accum_s, s_out)
            chunk(lambda: accum_v(v0, ev0, 0, E), u0_out)
            chunk(lambda: accum_v(v1, ev1, 0, E1A), u1a_out)

        @pl.when(cid == 1)
        def _():
            chunk(lambda: accum_v(v1, ev1, E1A, E - E1A), u1b_out)
            chunk(lambda: accum_v(v2, ev2, 0, E), u2_out)

    return sc_kernel


def kernel(s, v, edge_index, edge_dist, edge_vector, W1, b1, W2, b2, Wf, bf):
    N, H = s.shape
    E = edge_index.shape[1]
    src = edge_index[0]
    dst = edge_index[1]

    h0, h12 = _mlp_pallas(s, W1.T, b1.reshape(1, H), W2.T, b2.reshape(1, 3 * H))
    w0, w12 = _filter_pallas(edge_dist, Wf.T, bf.reshape(1, 3 * H))

    v0 = v[:, 0, :]
    v1 = v[:, 1, :]
    v2 = v[:, 2, :]
    ev0 = edge_vector[:, 0]
    ev1 = edge_vector[:, 1]
    ev2 = edge_vector[:, 2]

    s_out, u0, u1a, u1b, u2 = _make_sc_kernel(N, E, H)(
        h0, h12, w0, w12, v0, v1, v2, ev0, ev1, ev2, src, dst)
    u1 = _add_pallas(u1a, u1b)
    v_out = jnp.stack([u0, u1, u2], axis=1)
    return (s_out, v_out)


# speedup vs baseline: 1.3397x; 1.3397x over previous
import functools

import jax
import jax.numpy as jnp
from jax import lax
from jax.experimental import pallas as pl
from jax.experimental.pallas import tpu as pltpu
from jax.experimental.pallas import tpu_sc as plsc

L = 16
NTILES = 16
NB = 16
NR = 4
ZR = 8


def _mlp_pallas(s, W1t, b1, W2t, b2):
    N, H = s.shape
    TH = W2t.shape[1]
    R = 400
    assert N % R == 0

    def body(s_ref, w1_ref, b1_ref, w2_ref, b2_ref, h0_ref, h12_ref):
        t = jnp.dot(s_ref[...], w1_ref[...], preferred_element_type=jnp.float32)
        t = jnp.maximum(t + b1_ref[...], 0.0)
        hh = jnp.dot(t, w2_ref[...], preferred_element_type=jnp.float32)
        hh = hh + b2_ref[...]
        h0_ref[...] = hh[:, :H]
        h12_ref[...] = hh[:, H:]

    return pl.pallas_call(
        body,
        grid=(N // R,),
        in_specs=[
            pl.BlockSpec((R, H), lambda i: (i, 0)),
            pl.BlockSpec((H, H), lambda i: (0, 0)),
            pl.BlockSpec((1, H), lambda i: (0, 0)),
            pl.BlockSpec((H, TH), lambda i: (0, 0)),
            pl.BlockSpec((1, TH), lambda i: (0, 0)),
        ],
        out_specs=[
            pl.BlockSpec((R, H), lambda i: (i, 0)),
            pl.BlockSpec((R, TH - H), lambda i: (i, 0)),
        ],
        out_shape=[
            jax.ShapeDtypeStruct((N, H), jnp.float32),
            jax.ShapeDtypeStruct((N, TH - H), jnp.float32),
        ],
    )(s, W1t, b1, W2t, b2)


def _filter_pallas(edge_dist, Wft, bf):
    E, G = edge_dist.shape
    TH = Wft.shape[1]
    H = TH // 3
    R = 2000
    assert E % R == 0

    def body(d_ref, wf_ref, bf_ref, w0_ref, w12_ref):
        ww = jnp.dot(d_ref[...], wf_ref[...], preferred_element_type=jnp.float32)
        ww = ww + bf_ref[...]
        w0_ref[...] = ww[:, :H]
        w12_ref[...] = ww[:, H:]

    return pl.pallas_call(
        body,
        grid=(E // R,),
        in_specs=[
            pl.BlockSpec((R, G), lambda i: (i, 0)),
            pl.BlockSpec((G, TH), lambda i: (0, 0)),
            pl.BlockSpec((1, TH), lambda i: (0, 0)),
        ],
        out_specs=[
            pl.BlockSpec((R, H), lambda i: (i, 0)),
            pl.BlockSpec((R, TH - H), lambda i: (i, 0)),
        ],
        out_shape=[
            jax.ShapeDtypeStruct((E, H), jnp.float32),
            jax.ShapeDtypeStruct((E, TH - H), jnp.float32),
        ],
    )(edge_dist, Wft, bf)


@functools.cache
def _make_sc_kernel(N, E, H):
    assert N % ZR == 0
    assert E % (NTILES * NB) == 0
    nch = N // ZR
    nch_pt = -(-nch // NTILES)
    ept = E // NTILES
    nbatch = ept // NB
    nslc = H // L
    mesh = plsc.VectorSubcoreMesh(core_axis_name="c", subcore_axis_name="s",
                                  num_cores=2, num_subcores=NTILES)

    NS2 = 2 * NR
    small_types = [
        pltpu.VMEM((NB,), jnp.int32),
        pltpu.VMEM((NB,), jnp.int32),
        pltpu.VMEM((NB,), jnp.float32),
        pltpu.SemaphoreType.DMA,
    ]
    big_types = [
        pltpu.VMEM((NB, 2 * H), jnp.float32),
        pltpu.VMEM((NB, 2 * H), jnp.float32),
        pltpu.VMEM((NB, H), jnp.float32),
        pltpu.SemaphoreType.DMA,
        pltpu.SemaphoreType.DMA,
    ]

    @functools.partial(
        pl.kernel,
        out_type=[jax.ShapeDtypeStruct((N, H), jnp.float32)] * 4,
        mesh=mesh,
        scratch_types=[
            pltpu.VMEM_SHARED((N, H), jnp.float32),
            pltpu.VMEM((ZR, H), jnp.float32),
        ] + small_types * NS2 + big_types * NR,
    )
    def sc_kernel(h0, h12, w0, w12, v0, v1, v2, ev0, ev1, ev2, src, dst,
                  s_out, u0_out, u1_out, u2_out,
                  acc, zbuf, *ringargs):
        smalls = [ringargs[i * 4:(i + 1) * 4] for i in range(NS2)]
        bigargs = ringargs[NS2 * 4:]
        bigs = [bigargs[i * 5:(i + 1) * 5] for i in range(NR)]
        cid = lax.axis_index("c")
        sid = lax.axis_index("s")
        zero16 = jnp.zeros((L,), jnp.float32)

        def fill_zbuf(i, c):
            for j in range(nslc):
                zbuf[i, pl.ds(L * j, L)] = zero16
            return c

        def zero_acc():
            lax.fori_loop(0, ZR, fill_zbuf, 0)

            def z(i, c):
                cidx = sid + NTILES * i

                @pl.when(cidx < nch)
                def _():
                    pltpu.sync_copy(zbuf, acc.at[pl.ds(cidx * ZR, ZR)])
                return c
            lax.fori_loop(0, nch_pt, z, 0)

        def writeout(out_hbm):
            def wlp(i, c):
                cidx = sid + NTILES * i

                @pl.when(cidx < nch)
                def _():
                    r0 = cidx * ZR
                    pltpu.sync_copy(acc.at[pl.ds(r0, ZR)], zbuf)
                    pltpu.sync_copy(zbuf, out_hbm.at[pl.ds(r0, ZR)])
                return c
            lax.fori_loop(0, nch_pt, wlp, 0)

        def accum_s():
            def issue_small(b, k):
                srcb, dstb, _, semS = smalls[k]
                base = sid * ept + b * NB
                pltpu.async_copy(dst.at[pl.ds(base, NB)], dstb, semS)
                pltpu.async_copy(src.at[pl.ds(base, NB)], srcb, semS)

            def wait_small(b, k):
                srcb, dstb, _, semS = smalls[k]
                base = sid * ept + b * NB
                pltpu.make_async_copy(dst.at[pl.ds(base, NB)], dstb, semS).wait()
                pltpu.make_async_copy(src.at[pl.ds(base, NB)], srcb, semS).wait()

            def issue_big(b, k):
                srcb, semB = smalls[k][0], bigs[k][3]
                vkbuf, wk = bigs[k][2], bigs[k + NR // 2][2]
                base = sid * ept + b * NB
                pltpu.async_copy(h0.at[srcb], vkbuf, semB)
                pltpu.async_copy(w0.at[pl.ds(base, NB)], wk, semB)

            def wait_big(b, k):
                srcb, semB = smalls[k][0], bigs[k][3]
                vkbuf, wk = bigs[k][2], bigs[k + NR // 2][2]
                base = sid * ept + b * NB
                pltpu.make_async_copy(h0.at[srcb], vkbuf, semB).wait()
                pltpu.make_async_copy(w0.at[pl.ds(base, NB)], wk, semB).wait()

            def do_batch(b, k):
                dstb = smalls[k][1]
                vkbuf, wk = bigs[k][2], bigs[k + NR // 2][2]

                def edge(e, c2):
                    for j in range(nslc):
                        sl = pl.ds(L * j, L)
                        vkbuf[e, sl] = vkbuf[e, sl] * wk[e, sl]
                    return c2

                lax.fori_loop(0, NB, edge, 0)
                pltpu.sync_copy(vkbuf, acc.at[dstb], add=True)

            def pipelined_s():
                nrs = NR // 2
                for j in range(nrs):
                    issue_small(j, j)
                for j in range(nrs - 1):
                    wait_small(j, j)
                    issue_big(j, j)

                def phase(b, slot):
                    bn = b + nrs - 1
                    nslot = (slot + nrs - 1) % nrs

                    @pl.when(bn < nbatch)
                    def _():
                        wait_small(bn, nslot)
                        issue_big(bn, nslot)
                    wait_big(b, slot)
                    do_batch(b, slot)

                    @pl.when(b + nrs < nbatch)
                    def _():
                        issue_small(b + nrs, slot)

                def grp(i, c):
                    b0 = nrs * i
                    phase(b0, 0)
                    for p in range(1, nrs):
                        @pl.when(b0 + p < nbatch)
                        def _(p=p):
                            phase(b0 + p, p)
                    return c

                lax.fori_loop(0, -(-nbatch // nrs), grp, 0)

            pipelined_s()

        def accum_v(vk, evk):
            def issue_small(b, k):
                srcb, dstb, evb, semS = smalls[k]
                base = sid * ept + b * NB
                pltpu.async_copy(dst.at[pl.ds(base, NB)], dstb, semS)
                pltpu.async_copy(src.at[pl.ds(base, NB)], srcb, semS)
                pltpu.async_copy(evk.at[pl.ds(base, NB)], evb, semS)

            def wait_small(b, k):
                srcb, dstb, evb, semS = smalls[k]
                base = sid * ept + b * NB
                pltpu.make_async_copy(dst.at[pl.ds(base, NB)], dstb, semS).wait()
                pltpu.make_async_copy(src.at[pl.ds(base, NB)], srcb, semS).wait()
                pltpu.make_async_copy(evk.at[pl.ds(base, NB)], evb, semS).wait()

            def issue_big(b, k8, k4):
                srcb = smalls[k8][0]
                hbuf, wbuf, vkbuf, semB, _ = bigs[k4]
                base = sid * ept + b * NB
                pltpu.async_copy(h12.at[srcb], hbuf, semB)
                pltpu.async_copy(vk.at[srcb], vkbuf, semB)
                pltpu.async_copy(w12.at[pl.ds(base, NB)], wbuf, semB)

            def wait_big(b, k8, k4):
                srcb = smalls[k8][0]
                hbuf, wbuf, vkbuf, semB, _ = bigs[k4]
                base = sid * ept + b * NB
                pltpu.make_async_copy(h12.at[srcb], hbuf, semB).wait()
                pltpu.make_async_copy(vk.at[srcb], vkbuf, semB).wait()
                pltpu.make_async_copy(w12.at[pl.ds(base, NB)], wbuf, semB).wait()

            def wait_scatter(k8, k4):
                dstb = smalls[k8][1]
                vkbuf, semC = bigs[k4][2], bigs[k4][4]
                pltpu.make_async_copy(vkbuf, acc.at[dstb], semC).wait()

            def do_batch(b, k8, k4):
                dstb, evb = smalls[k8][1], smalls[k8][2]
                hbuf, wbuf, vkbuf, _, semC = bigs[k4]
                ev16 = evb[pl.ds(0, L)]

                def jbody(j, c3):
                    sl = pl.ds(L * j, L)
                    sl2 = pl.ds(H + L * j, L)
                    for t in range(L):
                        g1 = hbuf[t, sl] * wbuf[t, sl]
                        g2 = hbuf[t, sl2] * wbuf[t, sl2]
                        vkbuf[t, sl] = vkbuf[t, sl] * g1 + ev16[t] * g2
                    return c3

                lax.fori_loop(0, nslc, jbody, 0)
                pltpu.async_copy(vkbuf, acc.at[dstb], semC, add=True)

            for j in range(NS2 - 1):
                if j < nbatch:
                    issue_small(j, j)
            for j in range(NR - 1):
                if j < nbatch:
                    wait_small(j, j)
                    issue_big(j, j, j)

            def phase(b, p):
                p8 = p % NS2
                p4 = p % NR
                bn = b + NR - 1
                np8 = (p + NR - 1) % NS2
                np4 = (p + NR - 1) % NR

                @pl.when(bn < nbatch)
                def _():
                    wait_small(bn, np8)

                    @pl.when(b >= 1)
                    def _():
                        wait_scatter((p + NS2 - 1) % NS2, np4)
                    issue_big(bn, np8, np4)
                wait_big(b, p8, p4)
                do_batch(b, p8, p4)

                @pl.when(b + NS2 - 1 < nbatch)
                def _():
                    issue_small(b + NS2 - 1, (p + NS2 - 1) % NS2)

            def grp(i, c):
                b0 = NS2 * i
                phase(b0, 0)
                for p in range(1, NS2):
                    @pl.when(b0 + p < nbatch)
                    def _(p=p):
                        phase(b0 + p, p)
                return c

            lax.fori_loop(0, -(-nbatch // NS2), grp, 0)
            for j in range(NR):
                x = nbatch - NR + j
                if x >= 0:
                    wait_scatter(x % NS2, x % NR)

        def chunk(accum_fn, out_hbm):
            zero_acc()
            plsc.subcore_barrier()
            accum_fn()
            plsc.subcore_barrier()
            writeout(out_hbm)
            plsc.subcore_barrier()

        @pl.when(cid == 0)
        def _():
            chunk(accum_s, s_out)
            chunk(lambda: accum_v(v0, ev0), u0_out)

        @pl.when(cid == 1)
        def _():
            chunk(lambda: accum_v(v1, ev1), u1_out)
            chunk(lambda: accum_v(v2, ev2), u2_out)

    return sc_kernel


def kernel(s, v, edge_index, edge_dist, edge_vector, W1, b1, W2, b2, Wf, bf):
    N, H = s.shape
    E = edge_index.shape[1]
    src = edge_index[0]
    dst = edge_index[1]

    h0, h12 = _mlp_pallas(s, W1.T, b1.reshape(1, H), W2.T, b2.reshape(1, 3 * H))
    w0, w12 = _filter_pallas(edge_dist, Wf.T, bf.reshape(1, 3 * H))

    v0 = v[:, 0, :]
    v1 = v[:, 1, :]
    v2 = v[:, 2, :]
    ev0 = edge_vector[:, 0]
    ev1 = edge_vector[:, 1]
    ev2 = edge_vector[:, 2]

    s_out, u0, u1, u2 = _make_sc_kernel(N, E, H)(
        h0, h12, w0, w12, v0, v1, v2, ev0, ev1, ev2, src, dst)
    v_out = jnp.stack([u0, u1, u2], axis=1)
    return (s_out, v_out)


# speedup vs baseline: 1.4137x; 1.0553x over previous
import functools

import jax
import jax.numpy as jnp
from jax import lax
from jax.experimental import pallas as pl
from jax.experimental.pallas import tpu as pltpu
from jax.experimental.pallas import tpu_sc as plsc

L = 16
NTILES = 16
NB = 16
NR = 4
ZR = 8


def _mlp_pallas(s, W1t, b1, W2t, b2):
    N, H = s.shape
    TH = W2t.shape[1]
    R = 400
    assert N % R == 0

    def body(s_ref, w1_ref, b1_ref, w2_ref, b2_ref, h0_ref, h12_ref):
        t = jnp.dot(s_ref[...], w1_ref[...], preferred_element_type=jnp.float32)
        t = jnp.maximum(t + b1_ref[...], 0.0)
        hh = jnp.dot(t, w2_ref[...], preferred_element_type=jnp.float32)
        hh = hh + b2_ref[...]
        h0_ref[...] = hh[:, :H]
        h12_ref[...] = hh[:, H:]

    return pl.pallas_call(
        body,
        grid=(N // R,),
        in_specs=[
            pl.BlockSpec((R, H), lambda i: (i, 0)),
            pl.BlockSpec((H, H), lambda i: (0, 0)),
            pl.BlockSpec((1, H), lambda i: (0, 0)),
            pl.BlockSpec((H, TH), lambda i: (0, 0)),
            pl.BlockSpec((1, TH), lambda i: (0, 0)),
        ],
        out_specs=[
            pl.BlockSpec((R, H), lambda i: (i, 0)),
            pl.BlockSpec((R, TH - H), lambda i: (i, 0)),
        ],
        out_shape=[
            jax.ShapeDtypeStruct((N, H), jnp.float32),
            jax.ShapeDtypeStruct((N, TH - H), jnp.float32),
        ],
    )(s, W1t, b1, W2t, b2)


def _filter_pallas(edge_dist, Wft, bf):
    E, G = edge_dist.shape
    TH = Wft.shape[1]
    H = TH // 3
    R = 2000
    assert E % R == 0

    def body(d_ref, wf_ref, bf_ref, w0_ref, w12_ref):
        ww = jnp.dot(d_ref[...], wf_ref[...], preferred_element_type=jnp.float32)
        ww = ww + bf_ref[...]
        w0_ref[...] = ww[:, :H]
        w12_ref[...] = ww[:, H:]

    return pl.pallas_call(
        body,
        grid=(E // R,),
        in_specs=[
            pl.BlockSpec((R, G), lambda i: (i, 0)),
            pl.BlockSpec((G, TH), lambda i: (0, 0)),
            pl.BlockSpec((1, TH), lambda i: (0, 0)),
        ],
        out_specs=[
            pl.BlockSpec((R, H), lambda i: (i, 0)),
            pl.BlockSpec((R, TH - H), lambda i: (i, 0)),
        ],
        out_shape=[
            jax.ShapeDtypeStruct((E, H), jnp.float32),
            jax.ShapeDtypeStruct((E, TH - H), jnp.float32),
        ],
    )(edge_dist, Wft, bf)


@functools.cache
def _make_sc_kernel(N, E, H):
    assert N % ZR == 0
    assert E % (NTILES * NB) == 0
    nch = N // ZR
    nch_pt = -(-nch // NTILES)
    ept = E // NTILES
    nbatch = ept // NB
    nslc = H // L
    mesh = plsc.VectorSubcoreMesh(core_axis_name="c", subcore_axis_name="s",
                                  num_cores=2, num_subcores=NTILES)

    NS2 = 2 * NR
    small_types = [
        pltpu.VMEM((NB,), jnp.int32),
        pltpu.VMEM((NB,), jnp.int32),
        pltpu.VMEM((NB,), jnp.float32),
        pltpu.SemaphoreType.DMA,
    ]
    big_types = [
        pltpu.VMEM((NB, 2 * H), jnp.float32),
        pltpu.VMEM((NB, 2 * H), jnp.float32),
        pltpu.VMEM((NB, H), jnp.float32),
        pltpu.SemaphoreType.DMA,
        pltpu.SemaphoreType.DMA,
    ]

    @functools.partial(
        pl.kernel,
        out_type=[jax.ShapeDtypeStruct((N, H), jnp.float32)] * 4,
        mesh=mesh,
        scratch_types=[
            pltpu.VMEM_SHARED((N, H), jnp.float32),
            pltpu.VMEM((ZR, H), jnp.float32),
        ] + small_types * NS2 + big_types * NR,
    )
    def sc_kernel(h0, h12, w0, w12, v0, v1, v2, ev0, ev1, ev2, src, dst,
                  s_out, u0_out, u1_out, u2_out,
                  acc, zbuf, *ringargs):
        smalls = [ringargs[i * 4:(i + 1) * 4] for i in range(NS2)]
        bigargs = ringargs[NS2 * 4:]
        bigs = [bigargs[i * 5:(i + 1) * 5] for i in range(NR)]
        cid = lax.axis_index("c")
        sid = lax.axis_index("s")
        zero16 = jnp.zeros((L,), jnp.float32)

        def fill_zbuf(i, c):
            for j in range(nslc):
                zbuf[i, pl.ds(L * j, L)] = zero16
            return c

        def zero_acc():
            lax.fori_loop(0, ZR, fill_zbuf, 0)

            def z(i, c):
                cidx = sid + NTILES * i

                @pl.when(cidx < nch)
                def _():
                    pltpu.sync_copy(zbuf, acc.at[pl.ds(cidx * ZR, ZR)])
                return c
            lax.fori_loop(0, nch_pt, z, 0)

        def writeout(out_hbm):
            def wlp(i, c):
                cidx = sid + NTILES * i

                @pl.when(cidx < nch)
                def _():
                    r0 = cidx * ZR
                    pltpu.sync_copy(acc.at[pl.ds(r0, ZR)], zbuf)
                    pltpu.sync_copy(zbuf, out_hbm.at[pl.ds(r0, ZR)])
                return c
            lax.fori_loop(0, nch_pt, wlp, 0)

        def accum_s():
            def issue_small(b, k):
                srcb, dstb, _, semS = smalls[k]
                base = sid * ept + b * NB
                pltpu.async_copy(dst.at[pl.ds(base, NB)], dstb, semS)
                pltpu.async_copy(src.at[pl.ds(base, NB)], srcb, semS)

            def wait_small(b, k):
                srcb, dstb, _, semS = smalls[k]
                base = sid * ept + b * NB
                pltpu.make_async_copy(dst.at[pl.ds(base, NB)], dstb, semS).wait()
                pltpu.make_async_copy(src.at[pl.ds(base, NB)], srcb, semS).wait()

            def issue_big(b, k4, k2):
                srcb, semB = smalls[k4][0], bigs[k2][3]
                vkbuf, wk = bigs[k2][2], bigs[k2 + 2][2]
                base = sid * ept + b * NB
                pltpu.async_copy(h0.at[srcb], vkbuf, semB)
                pltpu.async_copy(w0.at[pl.ds(base, NB)], wk, semB)

            def wait_big(b, k4, k2):
                srcb, semB = smalls[k4][0], bigs[k2][3]
                vkbuf, wk = bigs[k2][2], bigs[k2 + 2][2]
                base = sid * ept + b * NB
                pltpu.make_async_copy(h0.at[srcb], vkbuf, semB).wait()
                pltpu.make_async_copy(w0.at[pl.ds(base, NB)], wk, semB).wait()

            def wait_scatter(k4, k2):
                dstb = smalls[k4][1]
                vkbuf, semC = bigs[k2][2], bigs[k2][4]
                pltpu.make_async_copy(vkbuf, acc.at[dstb], semC).wait()

            def do_batch(b, k4, k2):
                dstb = smalls[k4][1]
                vkbuf, wk, semC = bigs[k2][2], bigs[k2 + 2][2], bigs[k2][4]

                def edge(e, c2):
                    for j in range(nslc):
                        sl = pl.ds(L * j, L)
                        vkbuf[e, sl] = vkbuf[e, sl] * wk[e, sl]
                    return c2

                lax.fori_loop(0, NB, edge, 0)
                pltpu.async_copy(vkbuf, acc.at[dstb], semC, add=True)

            for j in range(3):
                issue_small(j, j)
            wait_small(0, 0)
            issue_big(0, 0, 0)

            def phase(b, p):
                p4 = p % 4
                k2 = p % 2
                np4 = (p + 1) % 4

                @pl.when(b + 1 < nbatch)
                def _():
                    wait_small(b + 1, np4)

                    @pl.when(b >= 1)
                    def _():
                        wait_scatter((p + 3) % 4, 1 - k2)
                    issue_big(b + 1, np4, 1 - k2)
                wait_big(b, p4, k2)
                do_batch(b, p4, k2)

                @pl.when(b + 3 < nbatch)
                def _():
                    issue_small(b + 3, (p + 3) % 4)

            def grp(i, c):
                b0 = 4 * i
                phase(b0, 0)
                for p in range(1, 4):
                    @pl.when(b0 + p < nbatch)
                    def _(p=p):
                        phase(b0 + p, p)
                return c

            lax.fori_loop(0, -(-nbatch // 4), grp, 0)
            for j in range(2):
                x = nbatch - 2 + j
                wait_scatter(x % 4, x % 2)

        def accum_v(vk, evk):
            def issue_small(b, k):
                srcb, dstb, evb, semS = smalls[k]
                base = sid * ept + b * NB
                pltpu.async_copy(dst.at[pl.ds(base, NB)], dstb, semS)
                pltpu.async_copy(src.at[pl.ds(base, NB)], srcb, semS)
                pltpu.async_copy(evk.at[pl.ds(base, NB)], evb, semS)

            def wait_small(b, k):
                srcb, dstb, evb, semS = smalls[k]
                base = sid * ept + b * NB
                pltpu.make_async_copy(dst.at[pl.ds(base, NB)], dstb, semS).wait()
                pltpu.make_async_copy(src.at[pl.ds(base, NB)], srcb, semS).wait()
                pltpu.make_async_copy(evk.at[pl.ds(base, NB)], evb, semS).wait()

            def issue_big(b, k8, k4):
                srcb = smalls[k8][0]
                hbuf, wbuf, vkbuf, semB, _ = bigs[k4]
                base = sid * ept + b * NB
                pltpu.async_copy(h12.at[srcb], hbuf, semB)
                pltpu.async_copy(vk.at[srcb], vkbuf, semB)
                pltpu.async_copy(w12.at[pl.ds(base, NB)], wbuf, semB)

            def wait_big(b, k8, k4):
                srcb = smalls[k8][0]
                hbuf, wbuf, vkbuf, semB, _ = bigs[k4]
                base = sid * ept + b * NB
                pltpu.make_async_copy(h12.at[srcb], hbuf, semB).wait()
                pltpu.make_async_copy(vk.at[srcb], vkbuf, semB).wait()
                pltpu.make_async_copy(w12.at[pl.ds(base, NB)], wbuf, semB).wait()

            def wait_scatter(k8, k4):
                dstb = smalls[k8][1]
                vkbuf, semC = bigs[k4][2], bigs[k4][4]
                pltpu.make_async_copy(vkbuf, acc.at[dstb], semC).wait()

            def do_batch(b, k8, k4):
                dstb, evb = smalls[k8][1], smalls[k8][2]
                hbuf, wbuf, vkbuf, _, semC = bigs[k4]
                ev16 = evb[pl.ds(0, L)]

                def jbody(j, c3):
                    sl = pl.ds(L * j, L)
                    sl2 = pl.ds(H + L * j, L)
                    for t in range(L):
                        g1 = hbuf[t, sl] * wbuf[t, sl]
                        g2 = hbuf[t, sl2] * wbuf[t, sl2]
                        vkbuf[t, sl] = vkbuf[t, sl] * g1 + ev16[t] * g2
                    return c3

                lax.fori_loop(0, nslc, jbody, 0)
                pltpu.async_copy(vkbuf, acc.at[dstb], semC, add=True)

            for j in range(NS2 - 1):
                if j < nbatch:
                    issue_small(j, j)
            for j in range(NR - 1):
                if j < nbatch:
                    wait_small(j, j)
                    issue_big(j, j, j)

            def phase(b, p):
                p8 = p % NS2
                p4 = p % NR
                bn = b + NR - 1
                np8 = (p + NR - 1) % NS2
                np4 = (p + NR - 1) % NR

                @pl.when(bn < nbatch)
                def _():
                    wait_small(bn, np8)

                    @pl.when(b >= 1)
                    def _():
                        wait_scatter((p + NS2 - 1) % NS2, np4)
                    issue_big(bn, np8, np4)
                wait_big(b, p8, p4)
                do_batch(b, p8, p4)

                @pl.when(b + NS2 - 1 < nbatch)
                def _():
                    issue_small(b + NS2 - 1, (p + NS2 - 1) % NS2)

            def grp(i, c):
                b0 = NS2 * i
                phase(b0, 0)
                for p in range(1, NS2):
                    @pl.when(b0 + p < nbatch)
                    def _(p=p):
                        phase(b0 + p, p)
                return c

            lax.fori_loop(0, -(-nbatch // NS2), grp, 0)
            for j in range(NR):
                x = nbatch - NR + j
                if x >= 0:
                    wait_scatter(x % NS2, x % NR)

        def chunk(accum_fn, out_hbm):
            zero_acc()
            plsc.subcore_barrier()
            accum_fn()
            plsc.subcore_barrier()
            writeout(out_hbm)
            plsc.subcore_barrier()

        @pl.when(cid == 0)
        def _():
            chunk(accum_s, s_out)
            chunk(lambda: accum_v(v0, ev0), u0_out)

        @pl.when(cid == 1)
        def _():
            chunk(lambda: accum_v(v1, ev1), u1_out)
            chunk(lambda: accum_v(v2, ev2), u2_out)

    return sc_kernel


def kernel(s, v, edge_index, edge_dist, edge_vector, W1, b1, W2, b2, Wf, bf):
    N, H = s.shape
    E = edge_index.shape[1]
    src = edge_index[0]
    dst = edge_index[1]

    h0, h12 = _mlp_pallas(s, W1.T, b1.reshape(1, H), W2.T, b2.reshape(1, 3 * H))
    w0, w12 = _filter_pallas(edge_dist, Wf.T, bf.reshape(1, 3 * H))

    v0 = v[:, 0, :]
    v1 = v[:, 1, :]
    v2 = v[:, 2, :]
    ev0 = edge_vector[:, 0]
    ev1 = edge_vector[:, 1]
    ev2 = edge_vector[:, 2]

    s_out, u0, u1, u2 = _make_sc_kernel(N, E, H)(
        h0, h12, w0, w12, v0, v1, v2, ev0, ev1, ev2, src, dst)
    v_out = jnp.stack([u0, u1, u2], axis=1)
    return (s_out, v_out)
